# baseline (device time: 24528 ns/iter reference)
import jax
import jax.numpy as jnp
from jax import lax
from jax.experimental import pallas as pl
from jax.experimental.pallas import tpu as pltpu

N_DEV = 4


def kernel(x, W1, W2):
    m, k = x.shape
    hdim = W1.shape[1]
    n = W2.shape[1]
    mc = m // N_DEV
    NS = 3
    nh = n // NS

    def body(x_ref, w1_ref, w2_ref, out_ref,
             w1b_ref, w2b_ref, hbuf, pbuf,
             rs_sbuf, rs_rbuf, ag_sbuf, ag_rbuf,
             rs_sc_sbuf, rs_sc_rbuf, ag_sc_sbuf, ag_sc_rbuf,
             rs_send_sems, rs_recv_sems, ag_send_sems, ag_recv_sems,
             rs_sc_send_sems, rs_sc_recv_sems,
             ag_sc_send_sems, ag_sc_recv_sems):
        my = lax.axis_index("i")

        barrier_sem = pltpu.get_barrier_semaphore()
        for j in range(1, N_DEV):
            pl.semaphore_signal(
                barrier_sem, inc=1,
                device_id=(lax.rem(my + j, N_DEV),),
                device_id_type=pl.DeviceIdType.MESH,
            )

        w1b_ref[...] = w1_ref[...].astype(jnp.bfloat16)
        w2b_ref[:, :nh] = w2_ref[:, :nh].astype(jnp.bfloat16)
        assert NS * nh == n

        def quantize(vals):
            s = jnp.maximum(jnp.max(jnp.abs(vals)), 1e-20)
            q = jnp.clip(
                jnp.floor(vals * (127.0 / s) + 0.5), -127.0, 127.0
            ).astype(jnp.int8)
            return q, jnp.full((8, 128), s, jnp.float32)

        def send_pair(j, h, src_idx, data_sbuf, data_rbuf, sc_sbuf, sc_rbuf,
                      send_sems, recv_sems, sc_send_sems, sc_recv_sems):
            peer = lax.rem(my + j, N_DEV)
            slot = N_DEV - 1 - j
            out = []
            for src, dst, ssem, rsem in (
                (sc_sbuf.at[(j - 1) * NS + h], sc_rbuf.at[slot * NS + h],
                 sc_send_sems.at[(j - 1) * NS + h],
                 sc_recv_sems.at[slot * NS + h]),
                (data_sbuf.at[src_idx, :, pl.ds(h * nh, nh)],
                 data_rbuf.at[slot, :, pl.ds(h * nh, nh)],
                 send_sems.at[(j - 1) * NS + h],
                 recv_sems.at[slot * NS + h]),
            ):
                rdma = pltpu.make_async_remote_copy(
                    src_ref=src, dst_ref=dst, send_sem=ssem, recv_sem=rsem,
                    device_id=(peer,), device_id_type=pl.DeviceIdType.MESH,
                )
                rdma.start()
                out.append(rdma)
            return out

        hbuf[...] = jnp.maximum(
            jnp.dot(x_ref[...].astype(jnp.bfloat16), w1b_ref[...],
                    preferred_element_type=jnp.float32),
            0.0,
        ).astype(jnp.bfloat16)
        rdmas = []
        for h in range(NS):
            cols = pl.ds(h * nh, nh)
            if h > 0:
                w2b_ref[:, cols] = w2_ref[:, cols].astype(jnp.bfloat16)
            pbuf[:, cols] = jnp.dot(
                hbuf[...],
                w2b_ref[:, pl.ds(h * nh, nh)],
                preferred_element_type=jnp.float32,
            )
            for j in range(1, N_DEV):
                peer = lax.rem(my + j, N_DEV)
                q, sc = quantize(pbuf[pl.ds(peer * mc, mc), cols])
                rs_sbuf[j - 1, :, cols] = q
                rs_sc_sbuf[(j - 1) * NS + h] = sc
                if h == 0 and j == 1:
                    pl.semaphore_wait(barrier_sem, N_DEV - 1)
                rdmas += send_pair(j, h, j - 1, rs_sbuf, rs_rbuf, rs_sc_sbuf,
                                   rs_sc_rbuf, rs_send_sems, rs_recv_sems,
                                   rs_sc_send_sems, rs_sc_recv_sems)

        def wait_recv(dst, rsem):
            pltpu.make_async_remote_copy(
                src_ref=dst, dst_ref=dst, send_sem=rsem, recv_sem=rsem,
                device_id=(my,), device_id_type=pl.DeviceIdType.MESH,
            ).wait_recv()

        for h in range(NS):
            cols = pl.ds(h * nh, nh)
            red = pbuf[pl.ds(my * mc, mc), cols]
            for slot in (2, 1, 0):
                wait_recv(rs_sc_rbuf.at[slot * NS + h],
                          rs_sc_recv_sems.at[slot * NS + h])
                wait_recv(rs_rbuf.at[slot, :, cols],
                          rs_recv_sems.at[slot * NS + h])
                sc = rs_sc_rbuf[slot * NS + h, 0, 0] * (1.0 / 127.0)
                red = red + rs_rbuf[slot, :, cols].astype(jnp.float32) * sc
            q, sc = quantize(red)
            ag_sbuf[0, :, cols] = q
            for j in range(1, N_DEV):
                ag_sc_sbuf[(j - 1) * NS + h] = sc
                rdmas += send_pair(j, h, 0, ag_sbuf, ag_rbuf,
                                   ag_sc_sbuf, ag_sc_rbuf,
                                   ag_send_sems, ag_recv_sems,
                                   ag_sc_send_sems, ag_sc_recv_sems)
            out_ref[pl.ds(my * mc, mc), cols] = red.astype(jnp.bfloat16)

        for h in range(NS):
            cols = pl.ds(h * nh, nh)
            for slot in (2, 1, 0):
                wait_recv(ag_sc_rbuf.at[slot * NS + h],
                          ag_sc_recv_sems.at[slot * NS + h])
                wait_recv(ag_rbuf.at[slot, :, cols],
                          ag_recv_sems.at[slot * NS + h])
                sc = ag_sc_rbuf[slot * NS + h, 0, 0] * (1.0 / 127.0)
                owner = lax.rem(my + 1 + slot, N_DEV)
                out_ref[pl.ds(owner * mc, mc), cols] = (
                    ag_rbuf[slot, :, cols].astype(jnp.float32) * sc
                ).astype(jnp.bfloat16)

        for rdma in rdmas:
            rdma.wait_send()

    return pl.pallas_call(
        body,
        out_shape=jax.ShapeDtypeStruct((m, n), jnp.bfloat16),
        in_specs=[
            pl.BlockSpec(memory_space=pltpu.VMEM),
            pl.BlockSpec(memory_space=pltpu.VMEM),
            pl.BlockSpec(memory_space=pltpu.VMEM),
        ],
        out_specs=pl.BlockSpec(memory_space=pltpu.VMEM),
        scratch_shapes=[
            pltpu.VMEM(W1.shape, jnp.bfloat16),
            pltpu.VMEM(W2.shape, jnp.bfloat16),
            pltpu.VMEM((m, hdim), jnp.bfloat16),
            pltpu.VMEM((m, n), jnp.float32),
            pltpu.VMEM((N_DEV - 1, mc, n), jnp.int8),
            pltpu.VMEM((N_DEV - 1, mc, n), jnp.int8),
            pltpu.VMEM((1, mc, n), jnp.int8),
            pltpu.VMEM((N_DEV - 1, mc, n), jnp.int8),
            pltpu.VMEM((4 * (N_DEV - 1), 8, 128), jnp.float32),
            pltpu.VMEM((4 * (N_DEV - 1), 8, 128), jnp.float32),
            pltpu.VMEM((4 * (N_DEV - 1), 8, 128), jnp.float32),
            pltpu.VMEM((4 * (N_DEV - 1), 8, 128), jnp.float32),
            pltpu.SemaphoreType.DMA((4 * (N_DEV - 1),)),
            pltpu.SemaphoreType.DMA((4 * (N_DEV - 1),)),
            pltpu.SemaphoreType.DMA((4 * (N_DEV - 1),)),
            pltpu.SemaphoreType.DMA((4 * (N_DEV - 1),)),
            pltpu.SemaphoreType.DMA((4 * (N_DEV - 1),)),
            pltpu.SemaphoreType.DMA((4 * (N_DEV - 1),)),
            pltpu.SemaphoreType.DMA((4 * (N_DEV - 1),)),
            pltpu.SemaphoreType.DMA((4 * (N_DEV - 1),)),
        ],
        compiler_params=pltpu.CompilerParams(collective_id=0),
    )(x, W1, W2)


# device time: 23707 ns/iter; 1.0346x vs baseline; 1.0346x over previous
import jax
import jax.numpy as jnp
from jax import lax
from jax.experimental import pallas as pl
from jax.experimental.pallas import tpu as pltpu

N_DEV = 4


def kernel(x, W1, W2):
    m, k = x.shape
    hdim = W1.shape[1]
    n = W2.shape[1]
    mc = m // N_DEV
    nh = n // 2

    def body(x_ref, w1_ref, w2_ref, out_ref,
             w1b_ref, w2b_ref, hbuf, pbuf,
             rs_sbuf, rs_rbuf, ag_sbuf, ag_rbuf,
             rs_sc_sbuf, rs_sc_rbuf, ag_sc_sbuf, ag_sc_rbuf,
             rs_send_sems, rs_recv_sems, ag_send_sems, ag_recv_sems,
             rs_sc_send_sems, rs_sc_recv_sems,
             ag_sc_send_sems, ag_sc_recv_sems):
        my = lax.axis_index("i")

        barrier_sem = pltpu.get_barrier_semaphore()
        for j in range(1, N_DEV):
            pl.semaphore_signal(
                barrier_sem, inc=1,
                device_id=(lax.rem(my + j, N_DEV),),
                device_id_type=pl.DeviceIdType.MESH,
            )

        w1b_ref[...] = w1_ref[...].astype(jnp.bfloat16)
        w2b_ref[:, :nh] = w2_ref[:, :nh].astype(jnp.bfloat16)

        def quantize(vals):
            s = jnp.maximum(jnp.max(jnp.abs(vals)), 1e-20)
            q = jnp.clip(
                jnp.floor(vals * (127.0 / s) + 0.5), -127.0, 127.0
            ).astype(jnp.int8)
            return q, jnp.full((8, 128), s, jnp.float32)

        def send_pair(j, h, src_idx, data_sbuf, data_rbuf, sc_sbuf, sc_rbuf,
                      send_sems, recv_sems, sc_send_sems, sc_recv_sems):
            peer = lax.rem(my + j, N_DEV)
            slot = N_DEV - 1 - j
            out = []
            for src, dst, ssem, rsem in (
                (sc_sbuf.at[(j - 1) * 2 + h], sc_rbuf.at[slot * 2 + h],
                 sc_send_sems.at[(j - 1) * 2 + h],
                 sc_recv_sems.at[slot * 2 + h]),
                (data_sbuf.at[src_idx, :, pl.ds(h * nh, nh)],
                 data_rbuf.at[slot, :, pl.ds(h * nh, nh)],
                 send_sems.at[(j - 1) * 2 + h],
                 recv_sems.at[slot * 2 + h]),
            ):
                rdma = pltpu.make_async_remote_copy(
                    src_ref=src, dst_ref=dst, send_sem=ssem, recv_sem=rsem,
                    device_id=(peer,), device_id_type=pl.DeviceIdType.MESH,
                )
                rdma.start()
                out.append(rdma)
            return out

        hbuf[...] = jnp.maximum(
            jnp.dot(x_ref[...].astype(jnp.bfloat16), w1b_ref[...],
                    preferred_element_type=jnp.float32),
            0.0,
        ).astype(jnp.bfloat16)
        rdmas = []
        for h in range(2):
            cols = pl.ds(h * nh, nh)
            if h == 1:
                w2b_ref[:, nh:] = w2_ref[:, nh:].astype(jnp.bfloat16)
            pbuf[:, cols] = jnp.dot(
                hbuf[...],
                w2b_ref[:, pl.ds(h * nh, nh)],
                preferred_element_type=jnp.float32,
            )
            for j in range(1, N_DEV):
                peer = lax.rem(my + j, N_DEV)
                q, sc = quantize(pbuf[pl.ds(peer * mc, mc), cols])
                rs_sbuf[j - 1, :, cols] = q
                rs_sc_sbuf[(j - 1) * 2 + h] = sc
                if h == 0 and j == 1:
                    pl.semaphore_wait(barrier_sem, N_DEV - 1)
                rdmas += send_pair(j, h, j - 1, rs_sbuf, rs_rbuf, rs_sc_sbuf,
                                   rs_sc_rbuf, rs_send_sems, rs_recv_sems,
                                   rs_sc_send_sems, rs_sc_recv_sems)

        def wait_recv(dst, rsem):
            pltpu.make_async_remote_copy(
                src_ref=dst, dst_ref=dst, send_sem=rsem, recv_sem=rsem,
                device_id=(my,), device_id_type=pl.DeviceIdType.MESH,
            ).wait_recv()

        for h in range(2):
            cols = pl.ds(h * nh, nh)
            red = pbuf[pl.ds(my * mc, mc), cols]
            for slot in (2, 1, 0):
                wait_recv(rs_sc_rbuf.at[slot * 2 + h],
                          rs_sc_recv_sems.at[slot * 2 + h])
                wait_recv(rs_rbuf.at[slot, :, cols],
                          rs_recv_sems.at[slot * 2 + h])
                sc = rs_sc_rbuf[slot * 2 + h, 0, 0] * (1.0 / 127.0)
                red = red + rs_rbuf[slot, :, cols].astype(jnp.float32) * sc
            q, sc = quantize(red)
            ag_sbuf[0, :, cols] = q
            for j in range(1, N_DEV):
                ag_sc_sbuf[(j - 1) * 2 + h] = sc
                rdmas += send_pair(j, h, 0, ag_sbuf, ag_rbuf,
                                   ag_sc_sbuf, ag_sc_rbuf,
                                   ag_send_sems, ag_recv_sems,
                                   ag_sc_send_sems, ag_sc_recv_sems)
            out_ref[pl.ds(my * mc, mc), cols] = red.astype(jnp.bfloat16)

        for h in range(2):
            cols = pl.ds(h * nh, nh)
            for slot in (2, 1, 0):
                wait_recv(ag_sc_rbuf.at[slot * 2 + h],
                          ag_sc_recv_sems.at[slot * 2 + h])
                wait_recv(ag_rbuf.at[slot, :, cols],
                          ag_recv_sems.at[slot * 2 + h])
                sc = ag_sc_rbuf[slot * 2 + h, 0, 0] * (1.0 / 127.0)
                owner = lax.rem(my + 1 + slot, N_DEV)
                out_ref[pl.ds(owner * mc, mc), cols] = (
                    ag_rbuf[slot, :, cols].astype(jnp.float32) * sc
                ).astype(jnp.bfloat16)

        for rdma in rdmas:
            rdma.wait_send()

    return pl.pallas_call(
        body,
        out_shape=jax.ShapeDtypeStruct((m, n), jnp.bfloat16),
        in_specs=[
            pl.BlockSpec(memory_space=pltpu.VMEM),
            pl.BlockSpec(memory_space=pltpu.VMEM),
            pl.BlockSpec(memory_space=pltpu.VMEM),
        ],
        out_specs=pl.BlockSpec(memory_space=pltpu.VMEM),
        scratch_shapes=[
            pltpu.VMEM(W1.shape, jnp.bfloat16),
            pltpu.VMEM(W2.shape, jnp.bfloat16),
            pltpu.VMEM((m, hdim), jnp.bfloat16),
            pltpu.VMEM((m, n), jnp.float32),
            pltpu.VMEM((N_DEV - 1, mc, n), jnp.int8),
            pltpu.VMEM((N_DEV - 1, mc, n), jnp.int8),
            pltpu.VMEM((1, mc, n), jnp.int8),
            pltpu.VMEM((N_DEV - 1, mc, n), jnp.int8),
            pltpu.VMEM((2 * (N_DEV - 1), 8, 128), jnp.float32),
            pltpu.VMEM((2 * (N_DEV - 1), 8, 128), jnp.float32),
            pltpu.VMEM((2 * (N_DEV - 1), 8, 128), jnp.float32),
            pltpu.VMEM((2 * (N_DEV - 1), 8, 128), jnp.float32),
            pltpu.SemaphoreType.DMA((2 * (N_DEV - 1),)),
            pltpu.SemaphoreType.DMA((2 * (N_DEV - 1),)),
            pltpu.SemaphoreType.DMA((2 * (N_DEV - 1),)),
            pltpu.SemaphoreType.DMA((2 * (N_DEV - 1),)),
            pltpu.SemaphoreType.DMA((2 * (N_DEV - 1),)),
            pltpu.SemaphoreType.DMA((2 * (N_DEV - 1),)),
            pltpu.SemaphoreType.DMA((2 * (N_DEV - 1),)),
            pltpu.SemaphoreType.DMA((2 * (N_DEV - 1),)),
        ],
        compiler_params=pltpu.CompilerParams(collective_id=0),
    )(x, W1, W2)
